# baseline (speedup 1.0000x reference)
import jax
import jax.numpy as jnp
from jax.experimental import pallas as pl
from jax.experimental.pallas import tpu as pltpu


def _mm(a, b):
    return jax.lax.dot_general(
        a.astype(jnp.bfloat16), b.astype(jnp.bfloat16),
        (((1,), (0,)), ((), ())),
        preferred_element_type=jnp.float32)


def _xw1_body(x_ref, w1_ref, z1_ref):
    z1_ref[...] = _mm(x_ref[...], w1_ref[...]).astype(jnp.bfloat16)


def _layer1_body(adj_ref, z1_ref, b1_ref, w2_ref, z2_ref):
    pre = _mm(adj_ref[...], z1_ref[...]) + b1_ref[...]
    h = jnp.maximum(pre, 0.0).astype(jnp.bfloat16)
    z2_ref[...] = _mm(h, w2_ref[...]).astype(jnp.bfloat16)


def _layer2_body(adj_ref, z2_ref, b2_ref, wc_ref, bc_ref, out_ref, cls_ref):
    o = _mm(adj_ref[...], z2_ref[...]) + b2_ref[...]
    out_ref[...] = o
    cls_ref[...] = _mm(o, wc_ref[...]) + bc_ref[...]


def _lsm_body(out_ref, lsm_ref):
    o = out_ref[...]
    m = jnp.max(o, axis=0, keepdims=True)
    s = jnp.sum(jnp.exp(o - m), axis=0, keepdims=True)
    lsm_ref[...] = o - (m + jnp.log(s))


def kernel(x, adj, W1, b1, W2, b2, Wc, bc):
    n, nfeat = x.shape
    nhid = W1.shape[1]
    ncls = Wc.shape[1]
    split = 4576

    rb = 400 if n % 400 == 0 else n
    ng = n // rb

    z1 = pl.pallas_call(
        _xw1_body,
        out_shape=jax.ShapeDtypeStruct((n, nhid), jnp.bfloat16),
    )(x, W1)

    z2 = pl.pallas_call(
        _layer1_body,
        grid=(ng,),
        in_specs=[
            pl.BlockSpec((rb, n), lambda k: (k, 0)),
            pl.BlockSpec((n, nhid), lambda k: (0, 0)),
            pl.BlockSpec((1, nhid), lambda k: (0, 0)),
            pl.BlockSpec((nhid, nfeat), lambda k: (0, 0)),
        ],
        out_specs=pl.BlockSpec((rb, nfeat), lambda k: (k, 0)),
        out_shape=jax.ShapeDtypeStruct((n, nfeat), jnp.bfloat16),
    )(adj, z1, b1.reshape(1, -1), W2)

    out, cls = pl.pallas_call(
        _layer2_body,
        grid=(ng,),
        in_specs=[
            pl.BlockSpec((rb, n), lambda k: (k, 0)),
            pl.BlockSpec((n, nfeat), lambda k: (0, 0)),
            pl.BlockSpec((1, nfeat), lambda k: (0, 0)),
            pl.BlockSpec((nfeat, ncls), lambda k: (0, 0)),
            pl.BlockSpec((1, ncls), lambda k: (0, 0)),
        ],
        out_specs=[
            pl.BlockSpec((rb, nfeat), lambda k: (k, 0)),
            pl.BlockSpec((rb, ncls), lambda k: (k, 0)),
        ],
        out_shape=[
            jax.ShapeDtypeStruct((n, nfeat), jnp.float32),
            jax.ShapeDtypeStruct((n, ncls), jnp.float32),
        ],
    )(adj, z2, b2.reshape(1, -1), Wc, bc.reshape(1, -1))

    lsm = pl.pallas_call(
        _lsm_body,
        out_shape=jax.ShapeDtypeStruct((n, nfeat), jnp.float32),
    )(out)

    return (lsm, cls[:split], cls[split:])
